# baseline (speedup 1.0000x reference)
import jax
import jax.numpy as jnp
from jax.experimental import pallas as pl
from jax.experimental.pallas import tpu as pltpu

_NB = 5


def _split(v):
    hi = v.astype(jnp.bfloat16)
    lo = (v - hi.astype(jnp.float32)).astype(jnp.bfloat16)
    return hi, lo


def _bT_mm(B, x):
    hi, lo = _split(x)
    dn = (((0,), (0,)), ((), ()))
    return (jax.lax.dot_general(B, hi, dn, preferred_element_type=jnp.float32)
            + jax.lax.dot_general(B, lo, dn, preferred_element_type=jnp.float32))


def _b_mm(B, y):
    hi, lo = _split(y)
    dn = (((1,), (0,)), ((), ()))
    return (jax.lax.dot_general(B, hi, dn, preferred_element_type=jnp.float32)
            + jax.lax.dot_general(B, lo, dn, preferred_element_type=jnp.float32))


def _xw_mm(x, w):
    xh, xl = _split(x)
    wh, wl = _split(w)
    dn = (((1,), (0,)), ((), ()))
    return (jax.lax.dot_general(xh, wh, dn, preferred_element_type=jnp.float32)
            + jax.lax.dot_general(xh, wl, dn, preferred_element_type=jnp.float32)
            + jax.lax.dot_general(xl, wh, dn, preferred_element_type=jnp.float32))


def _body(x0_ref, b_ref, w0_ref, w1_ref, out0_ref, out1_ref,
          acc1_ref, acc2_ref, y_ref):
    i = pl.program_id(0)
    p = i // _NB
    B = b_ref[...]

    @pl.when(p == 0)
    def _pass0():
        @pl.when(i == 0)
        def _z():
            acc1_ref[...] = jnp.zeros_like(acc1_ref)
        acc1_ref[...] += _bT_mm(B, x0_ref[...])

    @pl.when(p == 1)
    def _pass1():
        @pl.when(i == _NB)
        def _y():
            y_ref[...] = _xw_mm(acc1_ref[...], w0_ref[...])
            acc2_ref[...] = jnp.zeros_like(acc2_ref)
        x0p = _b_mm(B, y_ref[...])
        acc2_ref[...] += _bT_mm(B, x0p)

    @pl.when(p == 2)
    def _pass2():
        @pl.when(i == 2 * _NB)
        def _y2():
            out1_ref[...] = acc2_ref[...]
            y_ref[...] = _xw_mm(acc2_ref[...], w1_ref[...])
        out0_ref[...] = _b_mm(B, y_ref[...])


def kernel(x_0, incidence_1, W0, W1):
    n_nodes, ch = x_0.shape
    n_edges = incidence_1.shape[1]
    rb = n_nodes // _NB
    b_bf16 = incidence_1.astype(jnp.bfloat16)
    return pl.pallas_call(
        _body,
        grid=(3 * _NB,),
        in_specs=[
            pl.BlockSpec((rb, ch), lambda i: (jnp.minimum(i, _NB - 1), 0)),
            pl.BlockSpec((rb, n_edges), lambda i: (i % _NB, 0)),
            pl.BlockSpec((ch, ch), lambda i: (0, 0)),
            pl.BlockSpec((ch, ch), lambda i: (0, 0)),
        ],
        out_specs=(
            pl.BlockSpec((rb, ch), lambda i: (jnp.maximum(i - 2 * _NB, 0), 0)),
            pl.BlockSpec((n_edges, ch), lambda i: (0, 0)),
        ),
        out_shape=(
            jax.ShapeDtypeStruct((n_nodes, ch), jnp.float32),
            jax.ShapeDtypeStruct((n_edges, ch), jnp.float32),
        ),
        scratch_shapes=[
            pltpu.VMEM((n_edges, ch), jnp.float32),
            pltpu.VMEM((n_edges, ch), jnp.float32),
            pltpu.VMEM((n_edges, ch), jnp.float32),
        ],
    )(x_0, b_bf16, W0, W1)


# speedup vs baseline: 1.4561x; 1.4561x over previous
import jax
import jax.numpy as jnp
from jax.experimental import pallas as pl
from jax.experimental.pallas import tpu as pltpu

_NB = 5


def _bT_mm(B, x):
    dn = (((0,), (0,)), ((), ()))
    return jax.lax.dot_general(B, x.astype(jnp.bfloat16), dn,
                               preferred_element_type=jnp.float32)


def _b_mm(B, y):
    dn = (((1,), (0,)), ((), ()))
    return jax.lax.dot_general(B, y.astype(jnp.bfloat16), dn,
                               preferred_element_type=jnp.float32)


def _xw_mm(x, w):
    xh = x.astype(jnp.bfloat16)
    xl = (x - xh.astype(jnp.float32)).astype(jnp.bfloat16)
    wh = w.astype(jnp.bfloat16)
    wl = (w - wh.astype(jnp.float32)).astype(jnp.bfloat16)
    dn = (((1,), (0,)), ((), ()))
    return (jax.lax.dot_general(xh, wh, dn, preferred_element_type=jnp.float32)
            + jax.lax.dot_general(xh, wl, dn, preferred_element_type=jnp.float32)
            + jax.lax.dot_general(xl, wh, dn, preferred_element_type=jnp.float32))


def _body(x0_ref, b_ref, w0_ref, w1_ref, out0_ref, out1_ref,
          bc_ref, acc1_ref, acc2_ref, y_ref):
    i = pl.program_id(0)
    p = i // _NB
    r = i % _NB
    rb = b_ref.shape[0]

    @pl.when(p == 0)
    def _pass0():
        @pl.when(i == 0)
        def _z():
            acc1_ref[...] = jnp.zeros_like(acc1_ref)
        Bblk = b_ref[...].astype(jnp.bfloat16)
        bc_ref[pl.ds(r * rb, rb), :] = Bblk
        acc1_ref[...] += _bT_mm(Bblk, x0_ref[...])

    @pl.when(p == 1)
    def _pass1():
        @pl.when(i == _NB)
        def _y():
            y_ref[...] = _xw_mm(acc1_ref[...], w0_ref[...])
            acc2_ref[...] = jnp.zeros_like(acc2_ref)
        Bblk = bc_ref[pl.ds(r * rb, rb), :]
        x0p = _b_mm(Bblk, y_ref[...])
        acc2_ref[...] += _bT_mm(Bblk, x0p)

    @pl.when(p == 2)
    def _pass2():
        @pl.when(i == 2 * _NB)
        def _y2():
            out1_ref[...] = acc2_ref[...]
            y_ref[...] = _xw_mm(acc2_ref[...], w1_ref[...])
        out0_ref[...] = _b_mm(bc_ref[pl.ds(r * rb, rb), :], y_ref[...])


def kernel(x_0, incidence_1, W0, W1):
    n_nodes, ch = x_0.shape
    n_edges = incidence_1.shape[1]
    rb = n_nodes // _NB
    return pl.pallas_call(
        _body,
        grid=(3 * _NB,),
        in_specs=[
            pl.BlockSpec((rb, ch), lambda i: (jnp.minimum(i, _NB - 1), 0)),
            pl.BlockSpec((rb, n_edges), lambda i: (jnp.minimum(i, _NB - 1), 0)),
            pl.BlockSpec((ch, ch), lambda i: (0, 0)),
            pl.BlockSpec((ch, ch), lambda i: (0, 0)),
        ],
        out_specs=(
            pl.BlockSpec((rb, ch), lambda i: (jnp.maximum(i - 2 * _NB, 0), 0)),
            pl.BlockSpec((n_edges, ch), lambda i: (0, 0)),
        ),
        out_shape=(
            jax.ShapeDtypeStruct((n_nodes, ch), jnp.float32),
            jax.ShapeDtypeStruct((n_edges, ch), jnp.float32),
        ),
        scratch_shapes=[
            pltpu.VMEM((n_nodes, n_edges), jnp.bfloat16),
            pltpu.VMEM((n_edges, ch), jnp.float32),
            pltpu.VMEM((n_edges, ch), jnp.float32),
            pltpu.VMEM((n_edges, ch), jnp.float32),
        ],
    )(x_0, incidence_1, W0, W1)
